# Initial kernel scaffold; baseline (speedup 1.0000x reference)
import jax
import jax.numpy as jnp
from jax.experimental import pallas as pl


def kernel(x, a):
    raise NotImplementedError("write your pallas kernel here")


# speedup vs baseline: 6.1297x; 6.1297x over previous
import jax
import jax.numpy as jnp
from jax.experimental import pallas as pl
from jax.experimental.pallas import tpu as pltpu

_VMEM_LIMIT_BYTES = 60 * 1024 * 1024


def _pick_tile(n, target):
    best = 128
    t = 128
    while t <= min(n, target):
        if n % t == 0:
            best = t
        t *= 2
    return best


def _hop1_kernel(a_ref, x_ref, x1_ref):
    x1_ref[...] = jnp.dot(a_ref[...], x_ref[...],
                          preferred_element_type=jnp.float32)


def _hop2_concat_kernel(a_ref, x_ref, x1_ref, o_ref):
    i = pl.program_id(0)
    ti = a_ref.shape[0]
    f = x_ref.shape[1]
    row = i * ti
    o_ref[:, :f] = x_ref[pl.ds(row, ti), :]
    o_ref[:, f:2 * f] = x1_ref[pl.ds(row, ti), :]
    o_ref[:, 2 * f:] = jnp.dot(a_ref[...], x1_ref[...],
                               preferred_element_type=jnp.float32)


def _hop1(a, x, tile):
    n, f = x.shape
    return pl.pallas_call(
        _hop1_kernel,
        out_shape=jax.ShapeDtypeStruct((n, f), jnp.float32),
        grid=(n // tile,),
        in_specs=[
            pl.BlockSpec((tile, n), lambda i: (i, 0)),
            pl.BlockSpec((n, f), lambda i: (0, 0)),
        ],
        out_specs=pl.BlockSpec((tile, f), lambda i: (i, 0)),
        compiler_params=pltpu.CompilerParams(
            dimension_semantics=("parallel",),
            vmem_limit_bytes=_VMEM_LIMIT_BYTES,
        ),
    )(a, x)


def _hop2_concat(a, x, x1, tile):
    n, f = x.shape
    return pl.pallas_call(
        _hop2_concat_kernel,
        out_shape=jax.ShapeDtypeStruct((n, 3 * f), jnp.float32),
        grid=(n // tile,),
        in_specs=[
            pl.BlockSpec((tile, n), lambda i: (i, 0)),
            pl.BlockSpec((n, f), lambda i: (0, 0)),
            pl.BlockSpec((n, f), lambda i: (0, 0)),
        ],
        out_specs=pl.BlockSpec((tile, 3 * f), lambda i: (i, 0)),
        compiler_params=pltpu.CompilerParams(
            dimension_semantics=("parallel",),
            vmem_limit_bytes=_VMEM_LIMIT_BYTES,
        ),
    )(a, x, x1)


def kernel(x, a):
    n, _ = x.shape
    tile = _pick_tile(n, 512)
    x1 = _hop1(a, x, tile)
    return _hop2_concat(a, x, x1, tile)
